# baseline (speedup 1.0000x reference)
import functools

import jax
import jax.numpy as jnp
from jax import lax
from jax.experimental import pallas as pl
from jax.experimental.pallas import tpu as pltpu
from jax.experimental.pallas import tpu_sc as plsc

HASH_BUCKETS = 1000000
DIM = 32
B, L = 4096, 200
NC, NS = 2, 16
NW = NC * NS
ROWS_PER_W = B // NW
POS_PER_W = ROWS_PER_W * L
TOK0 = 8
CHUNK = 128
NCHUNK = POS_PER_W // CHUNK
NVEC = L // 16
NBUF = 4
NGRP = NCHUNK // NBUF
RB = 512
NBLK = 31232 // RB


def _i32(v):
    return jnp.int32(v)


def _hash16(a, b, c, d):
    k257, k9973 = _i32(257), _i32(9973)
    kq, ks, k1000 = _i32(33555), _i32(25), _i32(1000)
    k537000, k65537 = _i32(537000), _i32(65537)
    qd = (d * kq) >> ks
    rd = d - qd * k1000
    qc = (c * kq) >> ks
    rc = c - qc * k1000
    tri = b + c * k257 + qd * k537000 + rd * k65537
    four = a + b * k257 + qc * k537000 + rc * k65537 + d * k9973
    for k in range(7, -1, -1):
        cst = _i32(1000000 << k)
        tri = jnp.where(tri >= cst, tri - cst, tri)
    for k in range(9, -1, -1):
        cst = _i32(1000000 << k)
        four = jnp.where(four >= cst, four - cst, four)
    return tri, four


_mesh = plsc.VectorSubcoreMesh(core_axis_name="c", subcore_axis_name="s")


@functools.partial(
    pl.kernel,
    out_type=(jax.ShapeDtypeStruct((HASH_BUCKETS * DIM // 128, 128), jnp.float32),
              jax.ShapeDtypeStruct((HASH_BUCKETS * DIM // 128, 128), jnp.float32)),
    mesh=_mesh,
    scratch_types=[
        pltpu.VMEM((RB, DIM), jnp.float32),
        pltpu.VMEM((RB * DIM // 128, 128), jnp.float32),
        pltpu.SemaphoreType.DMA,
    ],
)
def _conv(tri_hbm, four_hbm, otri_hbm, ofour_hbm, bufa, bufb, sem):
    wid = (lax.axis_index("s").astype(jnp.int32) * _i32(NC)
           + lax.axis_index("c").astype(jnp.int32))
    base = wid * _i32(31232) + _i32(32) * jnp.minimum(wid, _i32(18))

    def blk(src, dst, r0, nrows):
        r0 = pl.multiple_of(r0, 32)
        pltpu.sync_copy(src.at[pl.ds(r0, nrows), :],
                        bufa.at[pl.ds(0, nrows), :])

        def rp(r, carry):
            for u in range(4):
                s = r * _i32(4) + _i32(u)
                for h2 in range(2):
                    bufb[r, pl.ds(u * 32 + h2 * 16, 16)] = (
                        bufa[s, pl.ds(h2 * 16, 16)])
            return carry

        lax.fori_loop(_i32(0), _i32(nrows // 4), rp, 0)
        pltpu.sync_copy(bufb.at[pl.ds(0, nrows * DIM // 128), :],
                        dst.at[pl.ds(pl.multiple_of(r0 // _i32(4), 8),
                                     nrows * DIM // 128), :])

    def body(i, carry):
        blk(tri_hbm, otri_hbm, base + i * _i32(RB), RB)
        blk(four_hbm, ofour_hbm, base + i * _i32(RB), RB)
        return carry

    lax.fori_loop(_i32(0), _i32(NBLK), body, 0)

    @pl.when(wid < _i32(18))
    def _():
        blk(tri_hbm, otri_hbm, base + _i32(NBLK * RB), 32)
        blk(four_hbm, ofour_hbm, base + _i32(NBLK * RB), 32)


@functools.partial(
    pl.kernel,
    out_type=jax.ShapeDtypeStruct((B * L * DIM // 128, 128), jnp.float32),
    mesh=_mesh,
    compiler_params=pltpu.CompilerParams(use_tc_tiling_on_sc=False),
    scratch_types=[
        pltpu.VMEM((TOK0 + POS_PER_W,), jnp.int32),
        pltpu.VMEM((POS_PER_W,), jnp.int32),
        pltpu.VMEM((POS_PER_W,), jnp.int32),
        pltpu.VMEM((NBUF, CHUNK, DIM), jnp.float32),
        pltpu.VMEM((NBUF, CHUNK, DIM), jnp.float32),
        pltpu.VMEM((NBUF, CHUNK * DIM // 128, 128), jnp.float32),
    ] + [pltpu.SemaphoreType.DMA] * (3 * NBUF),
)
def _embed(ids_hbm, tri_hbm, four_hbm, out_hbm,
           tok, itri, ifour, rtri, rfour, obuf, *sems):
    tsem = sems[0:NBUF]
    fsem = sems[NBUF:2 * NBUF]
    osem = sems[2 * NBUF:3 * NBUF]
    wid = (lax.axis_index("s").astype(jnp.int32) * _i32(NC)
           + lax.axis_index("c").astype(jnp.int32))
    base = wid * _i32(POS_PER_W)

    tok[pl.ds(0, 16)] = jnp.zeros((16,), jnp.int32)
    pltpu.sync_copy(ids_hbm.at[pl.ds(base, POS_PER_W)],
                    tok.at[pl.ds(TOK0, POS_PER_W)])

    iota16 = lax.iota(jnp.int32, 16)

    def taps(p):
        d = tok[pl.ds(p, 16)]
        c = tok[pl.ds(p - _i32(1), 16)]
        b = tok[pl.ds(p - _i32(2), 16)]
        a = tok[pl.ds(p - _i32(3), 16)]
        return a, b, c, d

    def hash_row(r, carry):
        rb = r * _i32(L)
        a, b, c, d = taps(rb + _i32(TOK0))
        zero = jnp.zeros((16,), jnp.int32)
        a = jnp.where(iota16 >= _i32(3), a, zero)
        b = jnp.where(iota16 >= _i32(2), b, zero)
        c = jnp.where(iota16 >= _i32(1), c, zero)
        tri, four = _hash16(a, b, c, d)
        itri[pl.ds(rb, 16)] = tri
        ifour[pl.ds(rb, 16)] = four

        def body(k, carry2):
            off = rb + k * _i32(16)
            a, b, c, d = taps(off + _i32(TOK0))
            tri, four = _hash16(a, b, c, d)
            itri[pl.ds(off, 16)] = tri
            ifour[pl.ds(off, 16)] = four
            return carry2

        lax.fori_loop(_i32(1), _i32(NVEC), body, 0)
        off = rb + _i32(L - 16)
        a, b, c, d = taps(off + _i32(TOK0))
        tri, four = _hash16(a, b, c, d)
        itri[pl.ds(off, 16)] = tri
        ifour[pl.ds(off, 16)] = four
        return carry

    lax.fori_loop(_i32(0), _i32(ROWS_PER_W), hash_row, 0)

    OROWS = CHUNK * DIM // 128

    def out_rows(fb):
        return pl.multiple_of((base + fb) * _i32(DIM) // _i32(128), 8)

    def fire_gathers(fb, b):
        pltpu.async_copy(tri_hbm.at[itri.at[pl.ds(fb, CHUNK)]],
                         rtri.at[_i32(b)], tsem[b])
        pltpu.async_copy(four_hbm.at[ifour.at[pl.ds(fb, CHUNK)]],
                         rfour.at[_i32(b)], fsem[b])

    def wait_gathers(fb, b):
        pltpu.make_async_copy(tri_hbm.at[itri.at[pl.ds(fb, CHUNK)]],
                              rtri.at[_i32(b)], tsem[b]).wait()
        pltpu.make_async_copy(four_hbm.at[ifour.at[pl.ds(fb, CHUNK)]],
                              rfour.at[_i32(b)], fsem[b]).wait()

    def drain_out(fb, b):
        pltpu.make_async_copy(obuf.at[_i32(b)],
                              out_hbm.at[pl.ds(out_rows(fb), OROWS), :],
                              osem[b]).wait()

    for b in range(NBUF):
        fire_gathers(_i32(b * CHUNK), b)

    def group(g, carry):
        j0 = g * _i32(NBUF)
        for b in range(NBUF):
            jj = j0 + _i32(b)
            fb = jj * _i32(CHUNK)
            wait_gathers(fb, b)

            @pl.when(g > _i32(0))
            def _():
                drain_out(fb, b)

            def add_body(r, carry2):
                for u in range(4):
                    s = r * _i32(4) + _i32(u)
                    for h2 in range(2):
                        sl = pl.ds(h2 * 16, 16)
                        obuf[_i32(b), r, pl.ds(u * 32 + h2 * 16, 16)] = (
                            rtri[_i32(b), s, sl] + rfour[_i32(b), s, sl])
                return carry2

            lax.fori_loop(_i32(0), _i32(OROWS), add_body, 0)
            pltpu.async_copy(obuf.at[_i32(b)],
                             out_hbm.at[pl.ds(out_rows(fb), OROWS), :],
                             osem[b])

            @pl.when(jj + _i32(NBUF) < _i32(NCHUNK))
            def _():
                fire_gathers(fb + _i32(NBUF * CHUNK), b)

        return carry

    lax.fori_loop(_i32(0), _i32(NGRP), group, 0)
    for b in range(NBUF):
        drain_out(_i32((NGRP - 1) * NBUF + b) * _i32(CHUNK), b)


@functools.partial(
    pl.kernel,
    out_type=jax.ShapeDtypeStruct((B, L, DIM), jnp.float32),
    mesh=_mesh,
    scratch_types=[
        pltpu.VMEM((4 * L * DIM // 128, 128), jnp.float32),
        pltpu.VMEM((4, L, DIM), jnp.float32),
        pltpu.SemaphoreType.DMA,
    ],
)
def _expand(in_hbm, out_hbm, bufa, bufb, sem):
    wid = (lax.axis_index("s").astype(jnp.int32) * _i32(NC)
           + lax.axis_index("c").astype(jnp.int32))
    s_base = wid * _i32(ROWS_PER_W)
    QROWS = 4 * L * DIM // 128

    def quad(q, carry):
        s0 = s_base + q * _i32(4)
        r0 = pl.multiple_of(s0 * _i32(L * DIM // 128), 8)
        pltpu.sync_copy(in_hbm.at[pl.ds(r0, QROWS), :], bufa)

        def rp(a, carry2):
            for j in range(4):
                for v in range(4):
                    t = a * _i32(4) + _i32(v)
                    src = _i32(j * 50) + a
                    for h2 in range(2):
                        bufb[_i32(j), t, pl.ds(h2 * 16, 16)] = (
                            bufa[src, pl.ds(v * 32 + h2 * 16, 16)])
            return carry2

        lax.fori_loop(_i32(0), _i32(L // 4), rp, 0)
        pltpu.sync_copy(bufb, out_hbm.at[pl.ds(s0, 4)])
        return carry

    lax.fori_loop(_i32(0), _i32(ROWS_PER_W // 4), quad, 0)


def kernel(input_ids, trigram_w, fourgram_w):
    ids = input_ids.reshape(-1).astype(jnp.int32)
    tri_f, four_f = _conv(trigram_w, fourgram_w)
    summed = _embed(ids, tri_f.reshape(HASH_BUCKETS, DIM),
                    four_f.reshape(HASH_BUCKETS, DIM))
    return _expand(summed)


# speedup vs baseline: 1.3808x; 1.3808x over previous
import functools

import jax
import jax.numpy as jnp
from jax import lax
from jax.experimental import pallas as pl
from jax.experimental.pallas import tpu as pltpu
from jax.experimental.pallas import tpu_sc as plsc

HASH_BUCKETS = 1000000
DIM = 32
B, L = 4096, 200
NC, NS = 2, 16
NW = NC * NS
ROWS_PER_W = B // NW
POS_PER_W = ROWS_PER_W * L
TOK0 = 8
CHUNK = 128
NCHUNK = POS_PER_W // CHUNK
NVEC = L // 16
NBUF = 4
NGRP = NCHUNK // NBUF


def _i32(v):
    return jnp.int32(v)


def _hash16(a, b, c, d):
    k257, k9973 = _i32(257), _i32(9973)
    kq, ks, k1000 = _i32(33555), _i32(25), _i32(1000)
    k537000, k65537 = _i32(537000), _i32(65537)
    qd = (d * kq) >> ks
    rd = d - qd * k1000
    qc = (c * kq) >> ks
    rc = c - qc * k1000
    tri = b + c * k257 + qd * k537000 + rd * k65537
    four = a + b * k257 + qc * k537000 + rc * k65537 + d * k9973
    for k in range(7, -1, -1):
        cst = _i32(1000000 << k)
        tri = jnp.where(tri >= cst, tri - cst, tri)
    for k in range(9, -1, -1):
        cst = _i32(1000000 << k)
        four = jnp.where(four >= cst, four - cst, four)
    return tri, four


_mesh = plsc.VectorSubcoreMesh(core_axis_name="c", subcore_axis_name="s")


@functools.partial(
    pl.kernel,
    out_type=jax.ShapeDtypeStruct((B * L * DIM,), jnp.float32),
    mesh=_mesh,
    compiler_params=pltpu.CompilerParams(use_tc_tiling_on_sc=False),
    scratch_types=[
        pltpu.VMEM((TOK0 + POS_PER_W,), jnp.int32),
        pltpu.VMEM((POS_PER_W,), jnp.int32),
        pltpu.VMEM((POS_PER_W,), jnp.int32),
        pltpu.VMEM((NBUF, CHUNK, DIM), jnp.float32),
        pltpu.VMEM((NBUF, CHUNK, DIM), jnp.float32),
        pltpu.VMEM((NBUF * CHUNK * DIM,), jnp.float32),
    ] + [pltpu.SemaphoreType.DMA] * (3 * NBUF),
)
def _embed(ids_hbm, tri_hbm, four_hbm, out_hbm,
           tok, itri, ifour, rtri, rfour, obuf, *sems):
    tsem = sems[0:NBUF]
    fsem = sems[NBUF:2 * NBUF]
    osem = sems[2 * NBUF:3 * NBUF]
    wid = (lax.axis_index("s").astype(jnp.int32) * _i32(NC)
           + lax.axis_index("c").astype(jnp.int32))
    base = wid * _i32(POS_PER_W)

    tok[pl.ds(0, 16)] = jnp.zeros((16,), jnp.int32)
    pltpu.sync_copy(ids_hbm.at[pl.ds(base, POS_PER_W)],
                    tok.at[pl.ds(TOK0, POS_PER_W)])

    iota16 = lax.iota(jnp.int32, 16)

    def taps(p):
        d = tok[pl.ds(p, 16)]
        c = tok[pl.ds(p - _i32(1), 16)]
        b = tok[pl.ds(p - _i32(2), 16)]
        a = tok[pl.ds(p - _i32(3), 16)]
        return a, b, c, d

    def hash_row(r, carry):
        rb = r * _i32(L)
        a, b, c, d = taps(rb + _i32(TOK0))
        zero = jnp.zeros((16,), jnp.int32)
        a = jnp.where(iota16 >= _i32(3), a, zero)
        b = jnp.where(iota16 >= _i32(2), b, zero)
        c = jnp.where(iota16 >= _i32(1), c, zero)
        tri, four = _hash16(a, b, c, d)
        itri[pl.ds(rb, 16)] = tri
        ifour[pl.ds(rb, 16)] = four

        def body(k, carry2):
            off = rb + k * _i32(16)
            a, b, c, d = taps(off + _i32(TOK0))
            tri, four = _hash16(a, b, c, d)
            itri[pl.ds(off, 16)] = tri
            ifour[pl.ds(off, 16)] = four
            return carry2

        lax.fori_loop(_i32(1), _i32(NVEC), body, 0)
        off = rb + _i32(L - 16)
        a, b, c, d = taps(off + _i32(TOK0))
        tri, four = _hash16(a, b, c, d)
        itri[pl.ds(off, 16)] = tri
        ifour[pl.ds(off, 16)] = four
        return carry

    lax.fori_loop(_i32(0), _i32(ROWS_PER_W), hash_row, 0)

    def fire_gathers(fb, b):
        pltpu.async_copy(tri_hbm.at[itri.at[pl.ds(fb, CHUNK)]],
                         rtri.at[_i32(b)], tsem[b])
        pltpu.async_copy(four_hbm.at[ifour.at[pl.ds(fb, CHUNK)]],
                         rfour.at[_i32(b)], fsem[b])

    def wait_gathers(fb, b):
        pltpu.make_async_copy(tri_hbm.at[itri.at[pl.ds(fb, CHUNK)]],
                              rtri.at[_i32(b)], tsem[b]).wait()
        pltpu.make_async_copy(four_hbm.at[ifour.at[pl.ds(fb, CHUNK)]],
                              rfour.at[_i32(b)], fsem[b]).wait()

    CD = CHUNK * DIM

    def drain_out(fb, b):
        pltpu.make_async_copy(obuf.at[pl.ds(b * CD, CD)],
                              out_hbm.at[pl.ds((base + fb) * _i32(DIM), CD)],
                              osem[b]).wait()

    for b in range(NBUF):
        fire_gathers(_i32(b * CHUNK), b)

    def group(g, carry):
        j0 = g * _i32(NBUF)
        for b in range(NBUF):
            jj = j0 + _i32(b)
            fb = jj * _i32(CHUNK)
            wait_gathers(fb, b)

            @pl.when(g > _i32(0))
            def _():
                drain_out(fb, b)

            def add_body(i, carry2):
                i8 = i * _i32(8)
                for u in range(8):
                    row = i8 + _i32(u)
                    opos = row * _i32(DIM) + _i32(b * CD)
                    for h in range(2):
                        sl = pl.ds(h * 16, 16)
                        obuf[pl.ds(opos + _i32(h * 16), 16)] = (
                            rtri[_i32(b), row, sl] + rfour[_i32(b), row, sl])
                return carry2

            lax.fori_loop(_i32(0), _i32(CHUNK // 8), add_body, 0)
            pltpu.async_copy(obuf.at[pl.ds(b * CD, CD)],
                             out_hbm.at[pl.ds((base + fb) * _i32(DIM), CD)],
                             osem[b])

            @pl.when(jj + _i32(NBUF) < _i32(NCHUNK))
            def _():
                fire_gathers(fb + _i32(NBUF * CHUNK), b)

        return carry

    lax.fori_loop(_i32(0), _i32(NGRP), group, 0)
    for b in range(NBUF):
        drain_out(_i32((NGRP - 1) * NBUF + b) * _i32(CHUNK), b)


@functools.partial(
    pl.kernel,
    out_type=jax.ShapeDtypeStruct((B, L, DIM), jnp.float32),
    mesh=_mesh,
    scratch_types=[
        pltpu.VMEM((4 * L * DIM,), jnp.float32),
        pltpu.VMEM((4, L, DIM), jnp.float32),
        pltpu.SemaphoreType.DMA,
    ],
)
def _expand(in_hbm, out_hbm, bufa, bufb, sem):
    wid = (lax.axis_index("s").astype(jnp.int32) * _i32(NC)
           + lax.axis_index("c").astype(jnp.int32))
    s_base = wid * _i32(ROWS_PER_W)
    QW = 4 * L * DIM

    def quad(q, carry):
        s0 = s_base + q * _i32(4)
        pltpu.sync_copy(in_hbm.at[pl.ds(s0 * _i32(L * DIM), QW)], bufa)

        def rp(a, carry2):
            for j in range(4):
                for v in range(4):
                    t = a * _i32(4) + _i32(v)
                    p = t * _i32(DIM) + _i32(j * L * DIM)
                    for h2 in range(2):
                        bufb[_i32(j), t, pl.ds(h2 * 16, 16)] = (
                            bufa[pl.ds(p + _i32(h2 * 16), 16)])
            return carry2

        lax.fori_loop(_i32(0), _i32(L // 4), rp, 0)
        pltpu.sync_copy(bufb, out_hbm.at[pl.ds(s0, 4)])
        return carry

    lax.fori_loop(_i32(0), _i32(ROWS_PER_W // 4), quad, 0)


def kernel(input_ids, trigram_w, fourgram_w):
    ids = input_ids.reshape(-1).astype(jnp.int32)
    flat = _embed(ids, trigram_w, fourgram_w)
    return _expand(flat)


# speedup vs baseline: 1.5879x; 1.1499x over previous
import functools

import jax
import jax.numpy as jnp
from jax import lax
from jax.experimental import pallas as pl
from jax.experimental.pallas import tpu as pltpu
from jax.experimental.pallas import tpu_sc as plsc

HASH_BUCKETS = 1000000
DIM = 32
B, L = 4096, 200
NC, NS = 2, 16
NW = NC * NS
ROWS_PER_W = B // NW
POS_PER_W = ROWS_PER_W * L
TOK0 = 8
CHUNK = 128
NCHUNK = POS_PER_W // CHUNK
NVEC = L // 16
NBUF = 4
NGRP = NCHUNK // NBUF


def _i32(v):
    return jnp.int32(v)


def _hash16(a, b, c, d):
    k257, k9973 = _i32(257), _i32(9973)
    kq, ks, k1000 = _i32(33555), _i32(25), _i32(1000)
    k537000, k65537 = _i32(537000), _i32(65537)
    qd = (d * kq) >> ks
    rd = d - qd * k1000
    qc = (c * kq) >> ks
    rc = c - qc * k1000
    tri = b + c * k257 + qd * k537000 + rd * k65537
    four = a + b * k257 + qc * k537000 + rc * k65537 + d * k9973
    for k in range(7, -1, -1):
        cst = _i32(1000000 << k)
        tri = jnp.where(tri >= cst, tri - cst, tri)
    for k in range(9, -1, -1):
        cst = _i32(1000000 << k)
        four = jnp.where(four >= cst, four - cst, four)
    return tri, four


_mesh = plsc.VectorSubcoreMesh(core_axis_name="c", subcore_axis_name="s")


@functools.partial(
    pl.kernel,
    out_type=jax.ShapeDtypeStruct((B * L * DIM,), jnp.float32),
    mesh=_mesh,
    compiler_params=pltpu.CompilerParams(use_tc_tiling_on_sc=False),
    scratch_types=[
        pltpu.VMEM((TOK0 + POS_PER_W,), jnp.int32),
        pltpu.VMEM((POS_PER_W,), jnp.int32),
        pltpu.VMEM((POS_PER_W,), jnp.int32),
        pltpu.VMEM((NBUF, CHUNK, DIM), jnp.float32),
        pltpu.VMEM((NBUF, CHUNK, DIM), jnp.float32),
        pltpu.VMEM((NBUF * CHUNK * DIM,), jnp.float32),
    ] + [pltpu.SemaphoreType.DMA] * (3 * NBUF),
)
def _embed(ids_hbm, tri_hbm, four_hbm, out_hbm,
           tok, itri, ifour, rtri, rfour, obuf, *sems):
    tsem = sems[0:NBUF]
    fsem = sems[NBUF:2 * NBUF]
    osem = sems[2 * NBUF:3 * NBUF]
    wid = (lax.axis_index("s").astype(jnp.int32) * _i32(NC)
           + lax.axis_index("c").astype(jnp.int32))
    base = wid * _i32(POS_PER_W)

    tok[pl.ds(0, 16)] = jnp.zeros((16,), jnp.int32)
    pltpu.sync_copy(ids_hbm.at[pl.ds(base, POS_PER_W)],
                    tok.at[pl.ds(TOK0, POS_PER_W)])

    iota16 = lax.iota(jnp.int32, 16)

    def taps(p):
        d = tok[pl.ds(p, 16)]
        c = tok[pl.ds(p - _i32(1), 16)]
        b = tok[pl.ds(p - _i32(2), 16)]
        a = tok[pl.ds(p - _i32(3), 16)]
        return a, b, c, d

    def hash_row(r, carry):
        rb = r * _i32(L)
        a, b, c, d = taps(rb + _i32(TOK0))
        zero = jnp.zeros((16,), jnp.int32)
        a = jnp.where(iota16 >= _i32(3), a, zero)
        b = jnp.where(iota16 >= _i32(2), b, zero)
        c = jnp.where(iota16 >= _i32(1), c, zero)
        tri, four = _hash16(a, b, c, d)
        itri[pl.ds(rb, 16)] = tri
        ifour[pl.ds(rb, 16)] = four

        def body(k, carry2):
            off = rb + k * _i32(16)
            a, b, c, d = taps(off + _i32(TOK0))
            tri, four = _hash16(a, b, c, d)
            itri[pl.ds(off, 16)] = tri
            ifour[pl.ds(off, 16)] = four
            return carry2

        lax.fori_loop(_i32(1), _i32(NVEC), body, 0)
        off = rb + _i32(L - 16)
        a, b, c, d = taps(off + _i32(TOK0))
        tri, four = _hash16(a, b, c, d)
        itri[pl.ds(off, 16)] = tri
        ifour[pl.ds(off, 16)] = four
        return carry

    lax.fori_loop(_i32(0), _i32(ROWS_PER_W), hash_row, 0)

    def fire_gathers(fb, b):
        pltpu.async_copy(tri_hbm.at[itri.at[pl.ds(fb, CHUNK)]],
                         rtri.at[_i32(b)], tsem[b])
        pltpu.async_copy(four_hbm.at[ifour.at[pl.ds(fb, CHUNK)]],
                         rfour.at[_i32(b)], fsem[b])

    def wait_gathers(fb, b):
        pltpu.make_async_copy(tri_hbm.at[itri.at[pl.ds(fb, CHUNK)]],
                              rtri.at[_i32(b)], tsem[b]).wait()
        pltpu.make_async_copy(four_hbm.at[ifour.at[pl.ds(fb, CHUNK)]],
                              rfour.at[_i32(b)], fsem[b]).wait()

    CD = CHUNK * DIM

    def drain_out(fb, b):
        pltpu.make_async_copy(obuf.at[pl.ds(b * CD, CD)],
                              out_hbm.at[pl.ds((base + fb) * _i32(DIM), CD)],
                              osem[b]).wait()

    for b in range(NBUF):
        fire_gathers(_i32(b * CHUNK), b)

    def group(g, carry):
        j0 = g * _i32(NBUF)
        for b in range(NBUF):
            jj = j0 + _i32(b)
            fb = jj * _i32(CHUNK)
            wait_gathers(fb, b)

            @pl.when(g > _i32(0))
            def _():
                drain_out(fb, b)

            def add_body(i, carry2):
                i8 = i * _i32(8)
                for u in range(8):
                    row = i8 + _i32(u)
                    opos = row * _i32(DIM) + _i32(b * CD)
                    for h in range(2):
                        sl = pl.ds(h * 16, 16)
                        obuf[pl.ds(opos + _i32(h * 16), 16)] = (
                            rtri[_i32(b), row, sl] + rfour[_i32(b), row, sl])
                return carry2

            lax.fori_loop(_i32(0), _i32(CHUNK // 8), add_body, 0)
            pltpu.async_copy(obuf.at[pl.ds(b * CD, CD)],
                             out_hbm.at[pl.ds((base + fb) * _i32(DIM), CD)],
                             osem[b])

            @pl.when(jj + _i32(NBUF) < _i32(NCHUNK))
            def _():
                fire_gathers(fb + _i32(NBUF * CHUNK), b)

        return carry

    lax.fori_loop(_i32(0), _i32(NGRP), group, 0)
    for b in range(NBUF):
        drain_out(_i32((NGRP - 1) * NBUF + b) * _i32(CHUNK), b)


def kernel(input_ids, trigram_w, fourgram_w):
    ids = input_ids.reshape(-1).astype(jnp.int32)
    out = _embed(ids, trigram_w, fourgram_w)
    return out.reshape(B, L, DIM)
